# baseline (speedup 1.0000x reference)
import functools

import jax
import jax.numpy as jnp
from jax import lax
from jax.experimental import pallas as pl
from jax.experimental.pallas import tpu as pltpu
from jax.experimental.pallas import tpu_sc as plsc

NC = 2
NS = 16
NW = NC * NS
L = 16

IDX_CHUNK = 128
ROW_UNROLL = 8


def _mf_kernel(b_per_w, n_chunks, n_feats):
    mesh = plsc.VectorSubcoreMesh(core_axis_name="c", subcore_axis_name="s")
    B = b_per_w * NW

    @functools.partial(
        pl.kernel,
        mesh=mesh,
        compiler_params=pltpu.CompilerParams(
            needs_layout_passes=False, use_tc_tiling_on_sc=False
        ),
        out_type=jax.ShapeDtypeStruct((B,), jnp.float32),
        scratch_types=[
            pltpu.VMEM((n_chunks, IDX_CHUNK), jnp.int32),
            pltpu.VMEM((n_chunks, IDX_CHUNK), jnp.int32),
            pltpu.VMEM((IDX_CHUNK, n_feats), jnp.float32),
            pltpu.VMEM((IDX_CHUNK, n_feats), jnp.float32),
            pltpu.VMEM((b_per_w,), jnp.float32),
            pltpu.VMEM((b_per_w,), jnp.float32),
            pltpu.VMEM((b_per_w,), jnp.float32),
            pltpu.SemaphoreType.DMA,
        ],
    )
    def k(user_hbm, item_hbm, ub_hbm, ib_hbm, ue_hbm, ie_hbm, out_hbm,
          uidx_v, iidx_v, urows_v, irows_v, ub_v, ib_v, out_v, sem):
        wid = lax.axis_index("s") * NC + lax.axis_index("c")
        base = wid * b_per_w

        pltpu.sync_copy(user_hbm.at[wid], uidx_v)
        pltpu.sync_copy(item_hbm.at[wid], iidx_v)

        for j in range(n_chunks):
            sl = pl.ds(j * IDX_CHUNK, IDX_CHUNK)
            cps = [
                pltpu.async_copy(ue_hbm.at[uidx_v.at[j]], urows_v, sem),
                pltpu.async_copy(ie_hbm.at[iidx_v.at[j]], irows_v, sem),
                pltpu.async_copy(ub_hbm.at[uidx_v.at[j]], ub_v.at[sl], sem),
                pltpu.async_copy(ib_hbm.at[iidx_v.at[j]], ib_v.at[sl], sem),
            ]
            for c in cps:
                c.wait()

            lane = lax.broadcasted_iota(jnp.int32, (L,), 0)

            def body(g, _, _j=j):
                acc = jnp.zeros((L,), jnp.float32)
                for r in range(L):
                    row = g * L + r
                    p = urows_v[row, :] * irows_v[row, :]
                    acc = jnp.where(lane == r, jnp.sum(p), acc)
                out_v[pl.ds(_j * IDX_CHUNK + g * L, L)] = acc
                return _

            lax.fori_loop(0, IDX_CHUNK // L, body, None)

        def bias_body(g, _):
            sl = pl.ds(g * L, L)
            out_v[sl] = out_v[sl] + ub_v[sl] + ib_v[sl]
            return _

        lax.fori_loop(0, b_per_w // L, bias_body, None)
        pltpu.sync_copy(out_v, out_hbm.at[pl.ds(base, b_per_w)])

    return k


def kernel(user, item, u_bias, i_bias, u_embed, i_embed):
    B = user.shape[0]
    n_feats = u_embed.shape[1]
    b_per_w = B // NW
    n_chunks = b_per_w // IDX_CHUNK

    user_r = user.astype(jnp.int32).reshape(NW, n_chunks, IDX_CHUNK)
    item_r = item.astype(jnp.int32).reshape(NW, n_chunks, IDX_CHUNK)
    ub = u_bias.reshape(-1)
    ib = i_bias.reshape(-1)

    k = _mf_kernel(b_per_w, n_chunks, n_feats)
    return k(user_r, item_r, ub, ib, u_embed, i_embed)
